# baseline (speedup 1.0000x reference)
import functools

import jax
import jax.numpy as jnp
from jax import lax
from jax.experimental import pallas as pl
from jax.experimental.pallas import tpu as pltpu
from jax.experimental.pallas import tpu_sc as plsc

K = 8192
D = 256
B, HW = 32, 576
M = B * HW

BM = 512
BK = 512
G = M // BM
NK = K // BK

NC, NS = 2, 16
NW = NC * NS
ROWS_PER_W = M // NW
GCH = 96
NCH = ROWS_PER_W // GCH


def _sqr_body(w_ref, sqr_ref):
    for kt in range(NK):
        w = w_ref[pl.ds(kt * BK, BK), :]
        sqr_ref[pl.ds(kt * BK, BK), :] = jnp.sum(w * w, axis=1, keepdims=True)


def _tc_sqr(weight):
    return pl.pallas_call(
        _sqr_body,
        in_specs=[pl.BlockSpec((K, D), lambda: (0, 0))],
        out_specs=pl.BlockSpec((K, 1), lambda: (0, 0)),
        out_shape=jax.ShapeDtypeStruct((K, 1), jnp.float32),
    )(weight)


def _argmin_body(x_ref, w_ref, sqr_ref, idx_ref):
    x = x_ref[...] * -2.0
    iota = lax.broadcasted_iota(jnp.int32, (BK, BM), 0).astype(jnp.float32)
    best = None
    besti = None
    for kt in range(NK):
        w = w_ref[pl.ds(kt * BK, BK), :]
        cov = lax.dot_general(
            w, x,
            dimension_numbers=(((1,), (1,)), ((), ())),
            preferred_element_type=jnp.float32,
            precision=lax.Precision.DEFAULT,
        )
        scores = cov + sqr_ref[pl.ds(kt * BK, BK), :]
        loc_min = jnp.min(scores, axis=0, keepdims=True)
        loc_arg = jnp.min(
            jnp.where(scores == loc_min, iota, float(K)), axis=0, keepdims=True
        ) + float(kt * BK)
        if kt == 0:
            best, besti = loc_min, loc_arg
        else:
            upd = loc_min < best
            besti = jnp.where(upd, loc_arg, besti)
            best = jnp.where(upd, loc_min, best)
    idx_ref[...] = besti.astype(jnp.int32).reshape(1, 1, BM)


def _tc_argmin(x2, weight, sqr, g):
    return pl.pallas_call(
        _argmin_body,
        grid=(g,),
        in_specs=[
            pl.BlockSpec((BM, D), lambda m: (m, 0)),
            pl.BlockSpec((K, D), lambda m: (0, 0)),
            pl.BlockSpec((K, 1), lambda m: (0, 0)),
        ],
        out_specs=pl.BlockSpec((1, 1, BM), lambda m: (m, 0, 0)),
        out_shape=jax.ShapeDtypeStruct((g, 1, BM), jnp.int32),
    )(x2, weight, sqr)


@functools.cache
def _sc_gather(offset, nrows):
    rows_per_w = nrows // NW
    nch = rows_per_w // GCH

    def body(idx_hbm, w_hbm, out_hbm, idx_all,
             rows0, rows1, gs0, gs1, ss0, ss1):
        wid = lax.axis_index("s") * NC + lax.axis_index("c")
        base = wid * rows_per_w
        pltpu.sync_copy(idx_hbm.at[pl.ds(base, rows_per_w)], idx_all)

        rows = (rows0, rows1)
        gsem = (gs0, gs1)
        ssem = (ss0, ss1)
        gath = [None, None]
        stor = [None, None]

        def start_gather(c):
            b = c % 2
            gath[b] = pltpu.async_copy(
                w_hbm.at[idx_all.at[pl.ds(c * GCH, GCH)]], rows[b], gsem[b])

        start_gather(0)
        for c in range(nch):
            b = c % 2
            if c + 1 < nch:
                if c >= 1:
                    stor[(c + 1) % 2].wait()
                start_gather(c + 1)
            gath[b].wait()
            stor[b] = pltpu.async_copy(
                rows[b], out_hbm.at[pl.ds(offset + base + c * GCH, GCH)],
                ssem[b])
        stor[0].wait()
        if nch > 1:
            stor[1].wait()

    return functools.partial(
        pl.kernel,
        out_type=jax.ShapeDtypeStruct((M, D), jnp.float32),
        mesh=plsc.VectorSubcoreMesh(core_axis_name="c", subcore_axis_name="s"),
        scratch_types=[
            pltpu.VMEM((rows_per_w,), jnp.int32),
            pltpu.VMEM((GCH, D), jnp.float32),
            pltpu.VMEM((GCH, D), jnp.float32),
            pltpu.SemaphoreType.DMA,
            pltpu.SemaphoreType.DMA,
            pltpu.SemaphoreType.DMA,
            pltpu.SemaphoreType.DMA,
        ],
    )(body)


def kernel(input, weight):
    x2 = input.reshape(M, D)
    sqr = _tc_sqr(weight)
    idx_flat = _tc_argmin(x2, weight, sqr, G).reshape(M)
    vectors = _sc_gather(0, M)(idx_flat, weight).reshape(B, HW, D)
    indices = idx_flat.reshape(B, HW)
    return vectors, indices, vectors


# speedup vs baseline: 1.0295x; 1.0295x over previous
import functools

import jax
import jax.numpy as jnp
from jax import lax
from jax.experimental import pallas as pl
from jax.experimental.pallas import tpu as pltpu
from jax.experimental.pallas import tpu_sc as plsc

K = 8192
D = 256
B, HW = 32, 576
M = B * HW

BM = 512
BK = 512
G = M // BM
NK = K // BK

NC, NS = 2, 16
NW = NC * NS
ROWS_PER_W = M // NW
GCH = 96
NCH = ROWS_PER_W // GCH


def _argmin_body(x_ref, w_ref, idx_ref, sqr_ref):
    @pl.when(pl.program_id(0) == 0)
    def _():
        for kt in range(NK):
            w = w_ref[pl.ds(kt * BK, BK), :]
            sqr_ref[pl.ds(kt * BK, BK), :] = jnp.sum(w * w, axis=1, keepdims=True)

    x = x_ref[...] * -2.0
    iota = lax.broadcasted_iota(jnp.int32, (BK, 1), 0).astype(jnp.float32)
    best = None
    besti = None
    for kt in range(NK):
        w = w_ref[pl.ds(kt * BK, BK), :]
        cov = lax.dot_general(
            w, x,
            dimension_numbers=(((1,), (1,)), ((), ())),
            preferred_element_type=jnp.float32,
            precision=lax.Precision.DEFAULT,
        )
        scores = cov + sqr_ref[pl.ds(kt * BK, BK), :]
        loc_min = jnp.min(scores, axis=0, keepdims=True)
        loc_arg = jnp.min(
            jnp.where(scores == loc_min, iota, float(K)), axis=0, keepdims=True
        ) + float(kt * BK)
        if kt == 0:
            best, besti = loc_min, loc_arg
        else:
            upd = loc_min < best
            besti = jnp.where(upd, loc_arg, besti)
            best = jnp.where(upd, loc_min, best)
    idx_ref[...] = besti.astype(jnp.int32).reshape(1, 1, BM)


def _tc_argmin(x2, weight, g):
    return pl.pallas_call(
        _argmin_body,
        grid=(g,),
        in_specs=[
            pl.BlockSpec((BM, D), lambda m: (m, 0)),
            pl.BlockSpec((K, D), lambda m: (0, 0)),
        ],
        out_specs=pl.BlockSpec((1, 1, BM), lambda m: (m, 0, 0)),
        out_shape=jax.ShapeDtypeStruct((g, 1, BM), jnp.int32),
        scratch_shapes=[pltpu.VMEM((K, 1), jnp.float32)],
    )(x2, weight)


@functools.cache
def _sc_gather(offset, nrows):
    rows_per_w = nrows // NW
    nch = rows_per_w // GCH

    def body(idx_hbm, w_hbm, out_hbm, idx_all,
             rows0, rows1, gs0, gs1, ss0, ss1):
        wid = lax.axis_index("s") * NC + lax.axis_index("c")
        base = wid * rows_per_w
        pltpu.sync_copy(idx_hbm.at[pl.ds(base, rows_per_w)], idx_all)

        rows = (rows0, rows1)
        gsem = (gs0, gs1)
        ssem = (ss0, ss1)
        gath = [None, None]
        stor = [None, None]

        def start_gather(c):
            b = c % 2
            gath[b] = pltpu.async_copy(
                w_hbm.at[idx_all.at[pl.ds(c * GCH, GCH)]], rows[b], gsem[b])

        start_gather(0)
        for c in range(nch):
            b = c % 2
            if c + 1 < nch:
                if c >= 1:
                    stor[(c + 1) % 2].wait()
                start_gather(c + 1)
            gath[b].wait()
            stor[b] = pltpu.async_copy(
                rows[b], out_hbm.at[pl.ds(offset + base + c * GCH, GCH)],
                ssem[b])
        stor[0].wait()
        if nch > 1:
            stor[1].wait()

    return functools.partial(
        pl.kernel,
        out_type=jax.ShapeDtypeStruct((M, D), jnp.float32),
        mesh=plsc.VectorSubcoreMesh(core_axis_name="c", subcore_axis_name="s"),
        scratch_types=[
            pltpu.VMEM((rows_per_w,), jnp.int32),
            pltpu.VMEM((GCH, D), jnp.float32),
            pltpu.VMEM((GCH, D), jnp.float32),
            pltpu.SemaphoreType.DMA,
            pltpu.SemaphoreType.DMA,
            pltpu.SemaphoreType.DMA,
            pltpu.SemaphoreType.DMA,
        ],
    )(body)


def kernel(input, weight):
    x2 = input.reshape(M, D)
    idx_flat = _tc_argmin(x2, weight, G).reshape(M)
    vectors = _sc_gather(0, M)(idx_flat, weight).reshape(B, HW, D)
    indices = idx_flat.reshape(B, HW)
    return vectors, indices, vectors


# speedup vs baseline: 1.2236x; 1.1885x over previous
import functools

import jax
import jax.numpy as jnp
from jax import lax
from jax.experimental import pallas as pl
from jax.experimental.pallas import tpu as pltpu
from jax.experimental.pallas import tpu_sc as plsc

K = 8192
D = 256
B, HW = 32, 576
M = B * HW

BM = 512
BK = 512
G = M // BM
NK = K // BK

NC, NS = 2, 16
NW = NC * NS
ROWS_PER_W = M // NW
GCH = 96
NCH = ROWS_PER_W // GCH


def _argmin_body(x_ref, w_ref, idx_ref, sqr_ref):
    @pl.when(pl.program_id(0) == 0)
    def _():
        for kt in range(NK):
            w = w_ref[pl.ds(kt * BK, BK), :]
            sqr_ref[pl.ds(kt * BK, BK), :] = jnp.sum(w * w, axis=1, keepdims=True)

    x = x_ref[...] * -2.0
    iota = lax.broadcasted_iota(jnp.int32, (BK, 1), 0).astype(jnp.float32)
    best = None
    besti = None
    for kt in range(NK):
        w = w_ref[pl.ds(kt * BK, BK), :]
        cov = lax.dot_general(
            w, x,
            dimension_numbers=(((1,), (1,)), ((), ())),
            preferred_element_type=jnp.float32,
            precision=lax.Precision.DEFAULT,
        )
        scores = cov + sqr_ref[pl.ds(kt * BK, BK), :]
        loc_min = jnp.min(scores, axis=0, keepdims=True)
        loc_arg = (jnp.argmin(scores, axis=0).astype(jnp.float32).reshape(1, BM)
                   + float(kt * BK))
        if kt == 0:
            best, besti = loc_min, loc_arg
        else:
            upd = loc_min < best
            besti = jnp.where(upd, loc_arg, besti)
            best = jnp.where(upd, loc_min, best)
    idx_ref[...] = besti.astype(jnp.int32).reshape(1, 1, BM)


def _tc_argmin(x2, weight, g):
    return pl.pallas_call(
        _argmin_body,
        grid=(g,),
        in_specs=[
            pl.BlockSpec((BM, D), lambda m: (m, 0)),
            pl.BlockSpec((K, D), lambda m: (0, 0)),
        ],
        out_specs=pl.BlockSpec((1, 1, BM), lambda m: (m, 0, 0)),
        out_shape=jax.ShapeDtypeStruct((g, 1, BM), jnp.int32),
        scratch_shapes=[pltpu.VMEM((K, 1), jnp.float32)],
    )(x2, weight)


@functools.cache
def _sc_gather(offset, nrows):
    rows_per_w = nrows // NW
    nch = rows_per_w // GCH

    def body(idx_hbm, w_hbm, out_hbm, idx_all,
             rows0, rows1, gs0, gs1, ss0, ss1):
        wid = lax.axis_index("s") * NC + lax.axis_index("c")
        base = wid * rows_per_w
        pltpu.sync_copy(idx_hbm.at[pl.ds(base, rows_per_w)], idx_all)

        rows = (rows0, rows1)
        gsem = (gs0, gs1)
        ssem = (ss0, ss1)
        gath = [None, None]
        stor = [None, None]

        def start_gather(c):
            b = c % 2
            gath[b] = pltpu.async_copy(
                w_hbm.at[idx_all.at[pl.ds(c * GCH, GCH)]], rows[b], gsem[b])

        start_gather(0)
        for c in range(nch):
            b = c % 2
            if c + 1 < nch:
                if c >= 1:
                    stor[(c + 1) % 2].wait()
                start_gather(c + 1)
            gath[b].wait()
            stor[b] = pltpu.async_copy(
                rows[b], out_hbm.at[pl.ds(offset + base + c * GCH, GCH)],
                ssem[b])
        stor[0].wait()
        if nch > 1:
            stor[1].wait()

    return functools.partial(
        pl.kernel,
        out_type=jax.ShapeDtypeStruct((M, D), jnp.float32),
        mesh=plsc.VectorSubcoreMesh(core_axis_name="c", subcore_axis_name="s"),
        scratch_types=[
            pltpu.VMEM((rows_per_w,), jnp.int32),
            pltpu.VMEM((GCH, D), jnp.float32),
            pltpu.VMEM((GCH, D), jnp.float32),
            pltpu.SemaphoreType.DMA,
            pltpu.SemaphoreType.DMA,
            pltpu.SemaphoreType.DMA,
            pltpu.SemaphoreType.DMA,
        ],
    )(body)


def kernel(input, weight):
    x2 = input.reshape(M, D)
    idx_flat = _tc_argmin(x2, weight, G).reshape(M)
    vectors = _sc_gather(0, M)(idx_flat, weight).reshape(B, HW, D)
    indices = idx_flat.reshape(B, HW)
    return vectors, indices, vectors


# speedup vs baseline: 1.2883x; 1.0529x over previous
import functools

import jax
import jax.numpy as jnp
from jax import lax
from jax.experimental import pallas as pl
from jax.experimental.pallas import tpu as pltpu
from jax.experimental.pallas import tpu_sc as plsc

K = 8192
D = 256
B, HW = 32, 576
M = B * HW

BM = 1024
BK = 512
G = M // BM
NK = K // BK

NC, NS = 2, 16
NW = NC * NS
ROWS_PER_W = M // NW
GCH = 96
NCH = ROWS_PER_W // GCH


def _argmin_body(x_ref, w_ref, idx_ref, sqr_ref):
    @pl.when(pl.program_id(0) == 0)
    def _():
        for kt in range(NK):
            w = w_ref[pl.ds(kt * BK, BK), :]
            sqr_ref[pl.ds(kt * BK, BK), :] = jnp.sum(w * w, axis=1, keepdims=True)

    x = x_ref[...] * -2.0
    best = None
    besti = None
    for kt in range(NK):
        w = w_ref[pl.ds(kt * BK, BK), :]
        cov = lax.dot_general(
            w, x,
            dimension_numbers=(((1,), (1,)), ((), ())),
            preferred_element_type=jnp.float32,
            precision=lax.Precision.DEFAULT,
        )
        scores = cov + sqr_ref[pl.ds(kt * BK, BK), :]
        loc_min = jnp.min(scores, axis=0, keepdims=True)
        loc_arg = (jnp.argmin(scores, axis=0).astype(jnp.float32).reshape(1, BM)
                   + float(kt * BK))
        if kt == 0:
            best, besti = loc_min, loc_arg
        else:
            upd = loc_min < best
            besti = jnp.where(upd, loc_arg, besti)
            best = jnp.where(upd, loc_min, best)
    idx_ref[...] = besti.astype(jnp.int32).reshape(1, 1, BM)


def _tc_argmin(x2, weight, g):
    return pl.pallas_call(
        _argmin_body,
        grid=(g,),
        in_specs=[
            pl.BlockSpec((BM, D), lambda m: (m, 0)),
            pl.BlockSpec((K, D), lambda m: (0, 0)),
        ],
        out_specs=pl.BlockSpec((1, 1, BM), lambda m: (m, 0, 0)),
        out_shape=jax.ShapeDtypeStruct((g, 1, BM), jnp.int32),
        scratch_shapes=[pltpu.VMEM((K, 1), jnp.float32)],
    )(x2, weight)


@functools.cache
def _sc_gather(offset, nrows):
    rows_per_w = nrows // NW
    nch = rows_per_w // GCH

    def body(idx_hbm, w_hbm, out_hbm, idx_all,
             rows0, rows1, gs0, gs1, ss0, ss1):
        wid = lax.axis_index("s") * NC + lax.axis_index("c")
        base = wid * rows_per_w
        pltpu.sync_copy(idx_hbm.at[pl.ds(base, rows_per_w)], idx_all)

        rows = (rows0, rows1)
        gsem = (gs0, gs1)
        ssem = (ss0, ss1)
        gath = [None, None]
        stor = [None, None]

        def start_gather(c):
            b = c % 2
            gath[b] = pltpu.async_copy(
                w_hbm.at[idx_all.at[pl.ds(c * GCH, GCH)]], rows[b], gsem[b])

        start_gather(0)
        for c in range(nch):
            b = c % 2
            if c + 1 < nch:
                if c >= 1:
                    stor[(c + 1) % 2].wait()
                start_gather(c + 1)
            gath[b].wait()
            stor[b] = pltpu.async_copy(
                rows[b], out_hbm.at[pl.ds(offset + base + c * GCH, GCH)],
                ssem[b])
        stor[0].wait()
        if nch > 1:
            stor[1].wait()

    return functools.partial(
        pl.kernel,
        out_type=jax.ShapeDtypeStruct((M, D), jnp.float32),
        mesh=plsc.VectorSubcoreMesh(core_axis_name="c", subcore_axis_name="s"),
        scratch_types=[
            pltpu.VMEM((rows_per_w,), jnp.int32),
            pltpu.VMEM((GCH, D), jnp.float32),
            pltpu.VMEM((GCH, D), jnp.float32),
            pltpu.SemaphoreType.DMA,
            pltpu.SemaphoreType.DMA,
            pltpu.SemaphoreType.DMA,
            pltpu.SemaphoreType.DMA,
        ],
    )(body)


def kernel(input, weight):
    x2 = input.reshape(M, D)
    idx_flat = _tc_argmin(x2, weight, G).reshape(M)
    vectors = _sc_gather(0, M)(idx_flat, weight).reshape(B, HW, D)
    indices = idx_flat.reshape(B, HW)
    return vectors, indices, vectors


# speedup vs baseline: 1.2929x; 1.0036x over previous
import functools

import jax
import jax.numpy as jnp
from jax import lax
from jax.experimental import pallas as pl
from jax.experimental.pallas import tpu as pltpu
from jax.experimental.pallas import tpu_sc as plsc

K = 8192
D = 256
B, HW = 32, 576
M = B * HW

BM = 2048
BK = 1024
G = M // BM
NK = K // BK

NC, NS = 2, 16
NW = NC * NS
ROWS_PER_W = M // NW
GCH = 96
NCH = ROWS_PER_W // GCH


def _argmin_body(x_ref, w_ref, idx_ref, sqr_ref):
    @pl.when(pl.program_id(0) == 0)
    def _():
        for kt in range(NK):
            w = w_ref[pl.ds(kt * BK, BK), :]
            sqr_ref[pl.ds(kt * BK, BK), :] = jnp.sum(w * w, axis=1, keepdims=True)

    x = x_ref[...] * -2.0
    best = None
    besti = None
    for kt in range(NK):
        w = w_ref[pl.ds(kt * BK, BK), :]
        cov = lax.dot_general(
            w, x,
            dimension_numbers=(((1,), (1,)), ((), ())),
            preferred_element_type=jnp.float32,
            precision=lax.Precision.DEFAULT,
        )
        scores = cov + sqr_ref[pl.ds(kt * BK, BK), :]
        loc_min = jnp.min(scores, axis=0, keepdims=True)
        loc_arg = (jnp.argmin(scores, axis=0).astype(jnp.float32).reshape(1, BM)
                   + float(kt * BK))
        if kt == 0:
            best, besti = loc_min, loc_arg
        else:
            upd = loc_min < best
            besti = jnp.where(upd, loc_arg, besti)
            best = jnp.where(upd, loc_min, best)
    idx_ref[...] = besti.astype(jnp.int32).reshape(1, 1, BM)


def _tc_argmin(x2, weight, g):
    return pl.pallas_call(
        _argmin_body,
        grid=(g,),
        in_specs=[
            pl.BlockSpec((BM, D), lambda m: (m, 0)),
            pl.BlockSpec((K, D), lambda m: (0, 0)),
        ],
        out_specs=pl.BlockSpec((1, 1, BM), lambda m: (m, 0, 0)),
        out_shape=jax.ShapeDtypeStruct((g, 1, BM), jnp.int32),
        scratch_shapes=[pltpu.VMEM((K, 1), jnp.float32)],
    )(x2, weight)


@functools.cache
def _sc_gather(offset, nrows):
    rows_per_w = nrows // NW
    nch = rows_per_w // GCH

    def body(idx_hbm, w_hbm, out_hbm, idx_all,
             rows0, rows1, gs0, gs1, ss0, ss1):
        wid = lax.axis_index("s") * NC + lax.axis_index("c")
        base = wid * rows_per_w
        pltpu.sync_copy(idx_hbm.at[pl.ds(base, rows_per_w)], idx_all)

        rows = (rows0, rows1)
        gsem = (gs0, gs1)
        ssem = (ss0, ss1)
        gath = [None, None]
        stor = [None, None]

        def start_gather(c):
            b = c % 2
            gath[b] = pltpu.async_copy(
                w_hbm.at[idx_all.at[pl.ds(c * GCH, GCH)]], rows[b], gsem[b])

        start_gather(0)
        for c in range(nch):
            b = c % 2
            if c + 1 < nch:
                if c >= 1:
                    stor[(c + 1) % 2].wait()
                start_gather(c + 1)
            gath[b].wait()
            stor[b] = pltpu.async_copy(
                rows[b], out_hbm.at[pl.ds(offset + base + c * GCH, GCH)],
                ssem[b])
        stor[0].wait()
        if nch > 1:
            stor[1].wait()

    return functools.partial(
        pl.kernel,
        out_type=jax.ShapeDtypeStruct((M, D), jnp.float32),
        mesh=plsc.VectorSubcoreMesh(core_axis_name="c", subcore_axis_name="s"),
        scratch_types=[
            pltpu.VMEM((rows_per_w,), jnp.int32),
            pltpu.VMEM((GCH, D), jnp.float32),
            pltpu.VMEM((GCH, D), jnp.float32),
            pltpu.SemaphoreType.DMA,
            pltpu.SemaphoreType.DMA,
            pltpu.SemaphoreType.DMA,
            pltpu.SemaphoreType.DMA,
        ],
    )(body)


def kernel(input, weight):
    x2 = input.reshape(M, D)
    idx_flat = _tc_argmin(x2, weight, G).reshape(M)
    vectors = _sc_gather(0, M)(idx_flat, weight).reshape(B, HW, D)
    indices = idx_flat.reshape(B, HW)
    return vectors, indices, vectors


# speedup vs baseline: 1.2988x; 1.0046x over previous
import functools

import jax
import jax.numpy as jnp
from jax import lax
from jax.experimental import pallas as pl
from jax.experimental.pallas import tpu as pltpu
from jax.experimental.pallas import tpu_sc as plsc

K = 8192
D = 256
B, HW = 32, 576
M = B * HW

BM = 2048
BK = 1024
G = M // BM
NK = K // BK

NC, NS = 2, 16
NW = NC * NS
ROWS_PER_W = M // NW
GCH = 96
NCH = ROWS_PER_W // GCH


def _argmin_body(x_ref, w_ref, idx_ref, sqr_ref):
    @pl.when(pl.program_id(0) == 0)
    def _():
        for kt in range(NK):
            w = w_ref[pl.ds(kt * BK, BK), :]
            sqr_ref[pl.ds(kt * BK, BK), :] = jnp.sum(w * w, axis=1, keepdims=True)

    x = x_ref[...] * -2.0
    best = None
    besti = None
    for kt in range(NK):
        w = w_ref[pl.ds(kt * BK, BK), :]
        cov = lax.dot_general(
            w, x,
            dimension_numbers=(((1,), (1,)), ((), ())),
            preferred_element_type=jnp.float32,
            precision=lax.Precision.DEFAULT,
        )
        scores = cov + sqr_ref[pl.ds(kt * BK, BK), :]
        loc_min = jnp.min(scores, axis=0, keepdims=True)
        loc_arg = (jnp.argmin(scores, axis=0).astype(jnp.float32).reshape(1, BM)
                   + float(kt * BK))
        if kt == 0:
            best, besti = loc_min, loc_arg
        else:
            upd = loc_min < best
            besti = jnp.where(upd, loc_arg, besti)
            best = jnp.where(upd, loc_min, best)
    idx_ref[...] = besti.astype(jnp.int32).reshape(1, 1, BM)


def _tc_argmin(x2, weight, g):
    return pl.pallas_call(
        _argmin_body,
        grid=(g,),
        in_specs=[
            pl.BlockSpec((BM, D), lambda m: (m, 0)),
            pl.BlockSpec((K, D), lambda m: (0, 0)),
        ],
        out_specs=pl.BlockSpec((1, 1, BM), lambda m: (m, 0, 0)),
        out_shape=jax.ShapeDtypeStruct((g, 1, BM), jnp.int32),
        scratch_shapes=[pltpu.VMEM((K, 1), jnp.float32)],
    )(x2, weight)


@functools.cache
def _sc_gather(offset, nrows):
    rows_per_w = nrows // NW
    nch = rows_per_w // GCH

    nbuf = 3

    def body(idx_hbm, w_hbm, out_hbm, idx_all, *bufs):
        rows = bufs[:nbuf]
        gsem = bufs[nbuf:2 * nbuf]
        ssem = bufs[2 * nbuf:]
        wid = lax.axis_index("s") * NC + lax.axis_index("c")
        base = wid * rows_per_w
        pltpu.sync_copy(idx_hbm.at[pl.ds(base, rows_per_w)], idx_all)

        gath = [None] * nbuf
        stor = [None] * nbuf

        def start_gather(c):
            b = c % nbuf
            gath[b] = pltpu.async_copy(
                w_hbm.at[idx_all.at[pl.ds(c * GCH, GCH)]], rows[b], gsem[b])

        for c in range(min(nbuf - 1, nch)):
            start_gather(c)
        for c in range(nch):
            b = c % nbuf
            nxt = c + nbuf - 1
            if nxt < nch:
                if c >= 1:
                    stor[nxt % nbuf].wait()
                start_gather(nxt)
            gath[b].wait()
            stor[b] = pltpu.async_copy(
                rows[b], out_hbm.at[pl.ds(offset + base + c * GCH, GCH)],
                ssem[b])
        for b in range(min(nbuf, nch)):
            stor[b].wait()

    return functools.partial(
        pl.kernel,
        out_type=jax.ShapeDtypeStruct((M, D), jnp.float32),
        mesh=plsc.VectorSubcoreMesh(core_axis_name="c", subcore_axis_name="s"),
        scratch_types=[
            pltpu.VMEM((rows_per_w,), jnp.int32),
            *[pltpu.VMEM((GCH, D), jnp.float32) for _ in range(nbuf)],
            *[pltpu.SemaphoreType.DMA for _ in range(2 * nbuf)],
        ],
    )(body)


def kernel(input, weight):
    x2 = input.reshape(M, D)
    idx_flat = _tc_argmin(x2, weight, G).reshape(M)
    vectors = _sc_gather(0, M)(idx_flat, weight).reshape(B, HW, D)
    indices = idx_flat.reshape(B, HW)
    return vectors, indices, vectors


# speedup vs baseline: 1.3064x; 1.0058x over previous
import functools

import jax
import jax.numpy as jnp
from jax import lax
from jax.experimental import pallas as pl
from jax.experimental.pallas import tpu as pltpu
from jax.experimental.pallas import tpu_sc as plsc

K = 8192
D = 256
B, HW = 32, 576
M = B * HW

BM = 2048
BK = 2048
G = M // BM
NK = K // BK

NC, NS = 2, 16
NW = NC * NS
ROWS_PER_W = M // NW
GCH = 96
NCH = ROWS_PER_W // GCH


def _argmin_body(x_ref, w_ref, idx_ref, sqr_ref):
    @pl.when(pl.program_id(0) == 0)
    def _():
        for kt in range(NK):
            w = w_ref[pl.ds(kt * BK, BK), :]
            sqr_ref[pl.ds(kt * BK, BK), :] = jnp.sum(w * w, axis=1, keepdims=True)

    x = x_ref[...] * -2.0
    best = None
    besti = None
    for kt in range(NK):
        w = w_ref[pl.ds(kt * BK, BK), :]
        cov = lax.dot_general(
            w, x,
            dimension_numbers=(((1,), (1,)), ((), ())),
            preferred_element_type=jnp.float32,
            precision=lax.Precision.DEFAULT,
        )
        scores = cov + sqr_ref[pl.ds(kt * BK, BK), :]
        loc_min = jnp.min(scores, axis=0, keepdims=True)
        loc_arg = (jnp.argmin(scores, axis=0).astype(jnp.float32).reshape(1, BM)
                   + float(kt * BK))
        if kt == 0:
            best, besti = loc_min, loc_arg
        else:
            upd = loc_min < best
            besti = jnp.where(upd, loc_arg, besti)
            best = jnp.where(upd, loc_min, best)
    idx_ref[...] = besti.astype(jnp.int32).reshape(1, 1, BM)


def _tc_argmin(x2, weight, g):
    return pl.pallas_call(
        _argmin_body,
        grid=(g,),
        in_specs=[
            pl.BlockSpec((BM, D), lambda m: (m, 0)),
            pl.BlockSpec((K, D), lambda m: (0, 0)),
        ],
        out_specs=pl.BlockSpec((1, 1, BM), lambda m: (m, 0, 0)),
        out_shape=jax.ShapeDtypeStruct((g, 1, BM), jnp.int32),
        scratch_shapes=[pltpu.VMEM((K, 1), jnp.float32)],
    )(x2, weight)


@functools.cache
def _sc_gather(offset, nrows):
    rows_per_w = nrows // NW
    nch = rows_per_w // GCH

    nbuf = 3

    def body(idx_hbm, w_hbm, out_hbm, idx_all, *bufs):
        rows = bufs[:nbuf]
        gsem = bufs[nbuf:2 * nbuf]
        ssem = bufs[2 * nbuf:]
        wid = lax.axis_index("s") * NC + lax.axis_index("c")
        base = wid * rows_per_w
        pltpu.sync_copy(idx_hbm.at[pl.ds(base, rows_per_w)], idx_all)

        gath = [None] * nbuf
        stor = [None] * nbuf

        def start_gather(c):
            b = c % nbuf
            gath[b] = pltpu.async_copy(
                w_hbm.at[idx_all.at[pl.ds(c * GCH, GCH)]], rows[b], gsem[b])

        for c in range(min(nbuf - 1, nch)):
            start_gather(c)
        for c in range(nch):
            b = c % nbuf
            nxt = c + nbuf - 1
            if nxt < nch:
                if c >= 1:
                    stor[nxt % nbuf].wait()
                start_gather(nxt)
            gath[b].wait()
            stor[b] = pltpu.async_copy(
                rows[b], out_hbm.at[pl.ds(offset + base + c * GCH, GCH)],
                ssem[b])
        for b in range(min(nbuf, nch)):
            stor[b].wait()

    return functools.partial(
        pl.kernel,
        out_type=jax.ShapeDtypeStruct((M, D), jnp.float32),
        mesh=plsc.VectorSubcoreMesh(core_axis_name="c", subcore_axis_name="s"),
        scratch_types=[
            pltpu.VMEM((rows_per_w,), jnp.int32),
            *[pltpu.VMEM((GCH, D), jnp.float32) for _ in range(nbuf)],
            *[pltpu.SemaphoreType.DMA for _ in range(2 * nbuf)],
        ],
    )(body)


def kernel(input, weight):
    x2 = input.reshape(M, D)
    idx_flat = _tc_argmin(x2, weight, G).reshape(M)
    vectors = _sc_gather(0, M)(idx_flat, weight).reshape(B, HW, D)
    indices = idx_flat.reshape(B, HW)
    return vectors, indices, vectors
